# Initial kernel scaffold; baseline (speedup 1.0000x reference)
import jax
import jax.numpy as jnp
from jax.experimental import pallas as pl


def kernel(depth, feat, ranks_depth, ranks_feat, ranks_bev, interval_starts, interval_lengths):
    raise NotImplementedError("write your pallas kernel here")


# speedup vs baseline: 5.8986x; 5.8986x over previous
import functools

import jax
import jax.numpy as jnp
from jax import lax
from jax.experimental import pallas as pl
from jax.experimental.pallas import tpu as pltpu
from jax.experimental.pallas import tpu_sc as plsc

B, N, D, fH, fW, C = 1, 6, 88, 16, 44, 64
DZ, DY, DX = 1, 128, 128
N_POINTS = 200000
N_ROWS = B * DZ * DY * DX

NC = 2
NS = 16
L = 16
BLK = 1024
GRP = 128
NG = BLK // GRP
CHUNK = N_POINTS // (NC * NS)
N_PAD = 201728

_mesh = plsc.VectorSubcoreMesh(core_axis_name="c", subcore_axis_name="s")


@functools.partial(
    pl.kernel,
    out_type=jax.ShapeDtypeStruct((NC, N_ROWS, C), jnp.float32),
    mesh=_mesh,
    compiler_params=pltpu.CompilerParams(use_tc_tiling_on_sc=False),
    scratch_types=[
        pltpu.VMEM_SHARED((N_ROWS, C), jnp.float32),
        pltpu.VMEM((GRP, C), jnp.float32),
        pltpu.VMEM((NG, GRP), jnp.int32),
        pltpu.VMEM((NG, GRP), jnp.int32),
        pltpu.VMEM((NG, GRP), jnp.int32),
        pltpu.VMEM((NG, GRP), jnp.float32),
        pltpu.VMEM((GRP, C), jnp.float32),
        pltpu.VMEM((GRP, C), jnp.float32),
        pltpu.SemaphoreType.DMA,
        pltpu.SemaphoreType.DMA,
    ],
)
def _bev_pool_sc(depth_hbm, feat_hbm, rd_hbm, rf_hbm, rb_hbm, out_hbm,
                 acc, zbuf, rb_v, rf_v, rd_v, dval_v, frows_v, w_v,
                 sem_a, sem_b):
    cid = lax.axis_index("c")
    sid = lax.axis_index("s")
    zero16 = jnp.zeros((L,), jnp.float32)

    for r in range(GRP):
        for c in range(C // L):
            zbuf[r, pl.ds(c * L, L)] = zero16
    rows_per_tile = N_ROWS // NS
    for k in range(rows_per_tile // GRP):
        zo = pl.multiple_of(sid * rows_per_tile + k * GRP, 8)
        pltpu.sync_copy(zbuf, acc.at[pl.ds(zo, GRP)])
    plsc.subcore_barrier()

    t_lo = (cid * NS + sid) * CHUNK
    t_hi = t_lo + CHUNK
    a_lo = pl.multiple_of((t_lo // 8) * 8, 8)
    nblk = (t_hi - a_lo + BLK - 1) // BLK

    lanes = lax.iota(jnp.int32, L)

    def block_body(j, _):
        base = pl.multiple_of(a_lo + j * BLK, 8)
        cps = []
        for i in range(NG):
            o = pl.multiple_of(base + i * GRP, 8)
            cps.append(pltpu.async_copy(
                rb_hbm.at[pl.ds(o, GRP)], rb_v.at[i], sem_a))
            cps.append(pltpu.async_copy(
                rf_hbm.at[pl.ds(o, GRP)], rf_v.at[i], sem_a))
            cps.append(pltpu.async_copy(
                rd_hbm.at[pl.ds(o, GRP)], rd_v.at[i], sem_a))
        for cp in cps:
            cp.wait()

        def grp_step(i, _):
            g_dep = pltpu.async_copy(depth_hbm.at[rd_v.at[i]],
                                     dval_v.at[i], sem_b)
            g_feat = pltpu.async_copy(feat_hbm.at[rf_v.at[i]],
                                      frows_v, sem_b)
            g_dep.wait()
            g_feat.wait()
            glob0 = base + i * GRP
            for g in range(GRP // L):
                d16 = dval_v[i, pl.ds(g * L, L)]
                idxv = glob0 + g * L + lanes
                mask = jnp.logical_and(idxv >= t_lo, idxv < t_hi)
                d16 = jnp.where(mask, d16, 0.0)
                for q in range(L):
                    w = d16[q] * jnp.ones((L,), jnp.float32)
                    r = g * L + q
                    for c in range(C // L):
                        w_v[r, pl.ds(c * L, L)] = (
                            w * frows_v[r, pl.ds(c * L, L)])
            pltpu.sync_copy(w_v, acc.at[rb_v.at[i]], add=True)
            return _

        lax.fori_loop(0, NG, grp_step, 0)
        return _

    lax.fori_loop(0, nblk, block_body, 0)

    plsc.subcore_barrier()

    for k in range(rows_per_tile // GRP):
        fo = pl.multiple_of(sid * rows_per_tile + k * GRP, 8)
        pltpu.sync_copy(acc.at[pl.ds(fo, GRP)],
                        out_hbm.at[cid, pl.ds(fo, GRP)])


def _add_planes_body(x_ref, o_ref):
    o_ref[...] = x_ref[0] + x_ref[1]


def _add_planes(x):
    return pl.pallas_call(
        _add_planes_body,
        grid=(8,),
        in_specs=[pl.BlockSpec((2, 1024, 128), lambda i: (0, i, 0))],
        out_specs=pl.BlockSpec((1024, 128), lambda i: (i, 0)),
        out_shape=jax.ShapeDtypeStruct((8192, 128), jnp.float32),
    )(x)


def kernel(depth, feat, ranks_depth, ranks_feat, ranks_bev,
           interval_starts, interval_lengths):
    depth_flat = depth.reshape(-1)
    feat_flat = feat.reshape(-1, C)
    pad = N_PAD - N_POINTS
    rd = jnp.pad(ranks_depth.astype(jnp.int32), (0, pad))
    rf = jnp.pad(ranks_feat.astype(jnp.int32), (0, pad))
    rb = jnp.pad(ranks_bev.astype(jnp.int32), (0, pad))
    acc2 = _bev_pool_sc(depth_flat, feat_flat, rd, rf, rb)
    out = _add_planes(acc2.reshape(NC, N_ROWS * C // 128, 128))
    bev = out.reshape(B, DZ, DY, DX, C)
    return jnp.transpose(bev, (0, 4, 1, 2, 3))


# speedup vs baseline: 7.5311x; 1.2768x over previous
import functools

import jax
import jax.numpy as jnp
from jax import lax
from jax.experimental import pallas as pl
from jax.experimental.pallas import tpu as pltpu
from jax.experimental.pallas import tpu_sc as plsc

B, N, D, fH, fW, C = 1, 6, 88, 16, 44, 64
DZ, DY, DX = 1, 128, 128
N_POINTS = 200000
N_ROWS = B * DZ * DY * DX

NC = 2
NS = 16
L = 16
BLK = 1024
GRP = 128
NG = BLK // GRP
CHUNK = N_POINTS // (NC * NS)
N_PAD = 201728

_mesh = plsc.VectorSubcoreMesh(core_axis_name="c", subcore_axis_name="s")


@functools.partial(
    pl.kernel,
    out_type=jax.ShapeDtypeStruct((NC, N_ROWS, C), jnp.float32),
    mesh=_mesh,
    compiler_params=pltpu.CompilerParams(use_tc_tiling_on_sc=False),
    scratch_types=[
        pltpu.VMEM_SHARED((N_ROWS, C), jnp.float32),
        pltpu.VMEM((64, C), jnp.float32),
        pltpu.VMEM((NG, GRP), jnp.int32),
        pltpu.VMEM((NG, GRP), jnp.int32),
        pltpu.VMEM((NG, GRP), jnp.int32),
        pltpu.VMEM((NG, GRP), jnp.float32),
        pltpu.VMEM((4, GRP, C), jnp.float32),
        pltpu.VMEM((2, GRP, C), jnp.float32),
        pltpu.SemaphoreType.DMA,
        pltpu.SemaphoreType.DMA,
        pltpu.SemaphoreType.DMA,
    ],
)
def _bev_pool_sc(depth_hbm, feat_hbm, rd_hbm, rf_hbm, rb_hbm, out_hbm,
                 acc, zbuf, rb_v, rf_v, rd_v, dval_v, frows_v, w_v,
                 sem_a, sem_b, sem_c):
    cid = lax.axis_index("c")
    sid = lax.axis_index("s")
    zero16 = jnp.zeros((L,), jnp.float32)

    for r in range(64):
        for c in range(C // L):
            zbuf[r, pl.ds(c * L, L)] = zero16
    rows_per_tile = N_ROWS // NS
    for k in range(rows_per_tile // 64):
        zo = pl.multiple_of(sid * rows_per_tile + k * 64, 8)
        pltpu.sync_copy(zbuf, acc.at[pl.ds(zo, 64)])
    plsc.subcore_barrier()

    t_lo = (cid * NS + sid) * CHUNK
    t_hi = t_lo + CHUNK
    a_lo = pl.multiple_of((t_lo // 8) * 8, 8)
    nblk = (t_hi - a_lo + BLK - 1) // BLK

    lanes = lax.iota(jnp.int32, L)

    def block_body(j, _):
        base = pl.multiple_of(a_lo + j * BLK, 8)
        cps = []
        for i in range(NG):
            o = pl.multiple_of(base + i * GRP, 8)
            cps.append(pltpu.async_copy(
                rb_hbm.at[pl.ds(o, GRP)], rb_v.at[i], sem_a))
            cps.append(pltpu.async_copy(
                rf_hbm.at[pl.ds(o, GRP)], rf_v.at[i], sem_a))
            cps.append(pltpu.async_copy(
                rd_hbm.at[pl.ds(o, GRP)], rd_v.at[i], sem_a))
        for cp in cps:
            cp.wait()

        gcps = [None] * NG
        for i in range(4):
            gcps[i] = (
                pltpu.async_copy(depth_hbm.at[rd_v.at[i]],
                                 dval_v.at[i], sem_b),
                pltpu.async_copy(feat_hbm.at[rf_v.at[i]],
                                 frows_v.at[i], sem_b))
        scat = [None, None]
        for i in range(NG):
            gcps[i][0].wait()
            gcps[i][1].wait()
            if scat[i % 2] is not None:
                scat[i % 2].wait()
            glob0 = base + i * GRP
            for g in range(GRP // L):
                d16 = dval_v[i, pl.ds(g * L, L)]
                idxv = glob0 + g * L + lanes
                mask = jnp.logical_and(idxv >= t_lo, idxv < t_hi)
                d16 = jnp.where(mask, d16, 0.0)
                for q in range(L):
                    w = d16[q] * jnp.ones((L,), jnp.float32)
                    r = g * L + q
                    for c in range(C // L):
                        w_v[i % 2, r, pl.ds(c * L, L)] = (
                            w * frows_v[i % 4, r, pl.ds(c * L, L)])
            scat[i % 2] = pltpu.async_copy(
                w_v.at[i % 2], acc.at[rb_v.at[i]], sem_c, add=True)
            if i + 4 < NG:
                gcps[i + 4] = (
                    pltpu.async_copy(depth_hbm.at[rd_v.at[i + 4]],
                                     dval_v.at[i + 4], sem_b),
                    pltpu.async_copy(feat_hbm.at[rf_v.at[i + 4]],
                                     frows_v.at[i % 4], sem_b))
        scat[0].wait()
        scat[1].wait()
        return _

    lax.fori_loop(0, nblk, block_body, 0)

    plsc.subcore_barrier()

    for k in range(rows_per_tile // GRP):
        fo = pl.multiple_of(sid * rows_per_tile + k * GRP, 8)
        pltpu.sync_copy(acc.at[pl.ds(fo, GRP)],
                        out_hbm.at[cid, pl.ds(fo, GRP)])


def _add_planes_body(x_ref, o_ref):
    o_ref[...] = x_ref[0] + x_ref[1]


def _add_planes(x):
    return pl.pallas_call(
        _add_planes_body,
        grid=(8,),
        in_specs=[pl.BlockSpec((2, 1024, 128), lambda i: (0, i, 0))],
        out_specs=pl.BlockSpec((1024, 128), lambda i: (i, 0)),
        out_shape=jax.ShapeDtypeStruct((8192, 128), jnp.float32),
    )(x)


def kernel(depth, feat, ranks_depth, ranks_feat, ranks_bev,
           interval_starts, interval_lengths):
    depth_flat = depth.reshape(-1)
    feat_flat = feat.reshape(-1, C)
    pad = N_PAD - N_POINTS
    rd = jnp.pad(ranks_depth.astype(jnp.int32), (0, pad))
    rf = jnp.pad(ranks_feat.astype(jnp.int32), (0, pad))
    rb = jnp.pad(ranks_bev.astype(jnp.int32), (0, pad))
    acc2 = _bev_pool_sc(depth_flat, feat_flat, rd, rf, rb)
    out = _add_planes(acc2.reshape(NC, N_ROWS * C // 128, 128))
    bev = out.reshape(B, DZ, DY, DX, C)
    return jnp.transpose(bev, (0, 4, 1, 2, 3))
